# Initial kernel scaffold; baseline (speedup 1.0000x reference)
import jax
import jax.numpy as jnp
from jax.experimental import pallas as pl


def kernel(hidden_states, position_ids, input_ln_w, q_w, k_w, v_w, o_w, q_ln_w, k_ln_w, post_ln_w, router_w, gate_w, up_w, down_w):
    raise NotImplementedError("write your pallas kernel here")


# speedup vs baseline: 1.3366x; 1.3366x over previous
import functools
import math

import jax
import jax.numpy as jnp
from jax.experimental import pallas as pl
from jax.experimental.pallas import tpu as pltpu

B, S, D = 1, 2048, 2048
H, KV, HD = 16, 4, 128
E, TOPK, FF = 8, 2, 768
EPS = 1e-6
THETA = 10000.0
NEG = -1e9

f32 = jnp.float32
bf16 = jnp.bfloat16


def _qkv_body(h_ref, lnw_ref, qw_ref, kw_ref, vw_ref, q_ref, k_ref, v_ref):
    x = h_ref[...]
    ms = jnp.mean(x * x, axis=1, keepdims=True)
    xn = x * jax.lax.rsqrt(ms + EPS) * lnw_ref[...]
    xb = xn.astype(bf16)
    q_ref[...] = jnp.dot(xb, qw_ref[...], preferred_element_type=f32)
    k_ref[...] = jnp.dot(xb, kw_ref[...], preferred_element_type=f32)
    v_ref[...] = jnp.dot(xb, vw_ref[...],
                         preferred_element_type=f32).astype(bf16)


def _qkv(h2d, input_ln_w, q_w, k_w, v_w, sb=256):
    n = S // sb
    return pl.pallas_call(
        _qkv_body,
        grid=(n,),
        in_specs=[
            pl.BlockSpec((sb, D), lambda i: (i, 0)),
            pl.BlockSpec((1, D), lambda i: (0, 0)),
            pl.BlockSpec((D, H * HD), lambda i: (0, 0)),
            pl.BlockSpec((D, KV * HD), lambda i: (0, 0)),
            pl.BlockSpec((D, KV * HD), lambda i: (0, 0)),
        ],
        out_specs=[
            pl.BlockSpec((sb, H * HD), lambda i: (i, 0)),
            pl.BlockSpec((sb, KV * HD), lambda i: (i, 0)),
            pl.BlockSpec((sb, KV * HD), lambda i: (i, 0)),
        ],
        out_shape=[
            jax.ShapeDtypeStruct((S, H * HD), f32),
            jax.ShapeDtypeStruct((S, KV * HD), f32),
            jax.ShapeDtypeStruct((S, KV * HD), bf16),
        ],
    )(h2d, input_ln_w.reshape(1, D), q_w.astype(bf16), k_w.astype(bf16),
      v_w.astype(bf16))


def _rot_cat(x):
    return jnp.concatenate([-x[:, HD // 2:], x[:, :HD // 2]], axis=1)


def _krope_body(k_ref, cos_ref, sin_ref, lnw_ref, o_ref):
    k = k_ref[...]
    ms = jnp.mean(k * k, axis=1, keepdims=True)
    kn = k * jax.lax.rsqrt(ms + EPS) * lnw_ref[...]
    o_ref[...] = (kn * cos_ref[...] + _rot_cat(kn) * sin_ref[...]).astype(bf16)


def _krope(k2d, cos_t, sin_t, k_ln_w, sb=512):
    n = S // sb
    return pl.pallas_call(
        _krope_body,
        grid=(KV, n),
        in_specs=[
            pl.BlockSpec((sb, HD), lambda kv, i: (i, kv)),
            pl.BlockSpec((sb, HD), lambda kv, i: (i, 0)),
            pl.BlockSpec((sb, HD), lambda kv, i: (i, 0)),
            pl.BlockSpec((1, HD), lambda kv, i: (0, 0)),
        ],
        out_specs=pl.BlockSpec((sb, HD), lambda kv, i: (i, kv)),
        out_shape=jax.ShapeDtypeStruct((S, KV * HD), bf16),
    )(k2d, cos_t, sin_t, k_ln_w.reshape(1, HD))


def _attn_body(q_ref, k_ref, v_ref, cos_ref, sin_ref, lnw_ref, o_ref, *, qb):
    i = pl.program_id(1)
    q = q_ref[...]
    ms = jnp.mean(q * q, axis=1, keepdims=True)
    qn = q * jax.lax.rsqrt(ms + EPS) * lnw_ref[...]
    qr = (qn * cos_ref[...] + _rot_cat(qn) * sin_ref[...]).astype(bf16)
    scores = jax.lax.dot_general(
        qr, k_ref[...], (((1,), (1,)), ((), ())),
        preferred_element_type=f32) * (1.0 / math.sqrt(HD))
    row = i * qb + jax.lax.broadcasted_iota(jnp.int32, (qb, S), 0)
    col = jax.lax.broadcasted_iota(jnp.int32, (qb, S), 1)
    scores = jnp.where(col <= row, scores, NEG)
    m = jnp.max(scores, axis=1, keepdims=True)
    p = jnp.exp(scores - m)
    attn = (p / jnp.sum(p, axis=1, keepdims=True)).astype(bf16)
    o_ref[...] = jnp.dot(attn, v_ref[...],
                         preferred_element_type=f32).astype(bf16)


def _attention(q2d, kr2d, v2d, cos_t, sin_t, q_ln_w, qb=256):
    n = S // qb
    return pl.pallas_call(
        functools.partial(_attn_body, qb=qb),
        grid=(H, n),
        in_specs=[
            pl.BlockSpec((qb, HD), lambda h, i: (i, h)),
            pl.BlockSpec((S, HD), lambda h, i: (0, h // (H // KV))),
            pl.BlockSpec((S, HD), lambda h, i: (0, h // (H // KV))),
            pl.BlockSpec((qb, HD), lambda h, i: (i, 0)),
            pl.BlockSpec((qb, HD), lambda h, i: (i, 0)),
            pl.BlockSpec((1, HD), lambda h, i: (0, 0)),
        ],
        out_specs=pl.BlockSpec((qb, HD), lambda h, i: (i, h)),
        out_shape=jax.ShapeDtypeStruct((S, H * HD), bf16),
    )(q2d, kr2d, v2d, cos_t, sin_t, q_ln_w.reshape(1, HD))


def _oproj_body(o_ref, ow_ref, hid_ref, lnw_ref, rw_ref, hs_ref, xn_ref,
                lg_ref):
    att = jnp.dot(o_ref[...], ow_ref[...], preferred_element_type=f32)
    hs = hid_ref[...] + att
    hs_ref[...] = hs
    ms = jnp.mean(hs * hs, axis=1, keepdims=True)
    xn = hs * jax.lax.rsqrt(ms + EPS) * lnw_ref[...]
    xb = xn.astype(bf16)
    xn_ref[...] = xb
    lg_ref[...] = jnp.dot(xb, rw_ref[...], preferred_element_type=f32)


def _oproj(o2d, o_w, hid2d, post_ln_w, rw_pad, sb=256):
    n = S // sb
    return pl.pallas_call(
        _oproj_body,
        grid=(n,),
        in_specs=[
            pl.BlockSpec((sb, H * HD), lambda i: (i, 0)),
            pl.BlockSpec((H * HD, D), lambda i: (0, 0)),
            pl.BlockSpec((sb, D), lambda i: (i, 0)),
            pl.BlockSpec((1, D), lambda i: (0, 0)),
            pl.BlockSpec((D, 128), lambda i: (0, 0)),
        ],
        out_specs=[
            pl.BlockSpec((sb, D), lambda i: (i, 0)),
            pl.BlockSpec((sb, D), lambda i: (i, 0)),
            pl.BlockSpec((sb, 128), lambda i: (i, 0)),
        ],
        out_shape=[
            jax.ShapeDtypeStruct((S, D), f32),
            jax.ShapeDtypeStruct((S, D), bf16),
            jax.ShapeDtypeStruct((S, 128), f32),
        ],
    )(o2d, o_w.astype(bf16), hid2d, post_ln_w.reshape(1, D), rw_pad)


def _route_body(lg_ref, cmb_ref):
    l = lg_ref[...]
    sb = l.shape[0]
    lane = jax.lax.broadcasted_iota(jnp.int32, (sb, 128), 1)
    valid = lane < E
    l = jnp.where(valid, l, -1e30)
    m0 = jnp.max(l, axis=1, keepdims=True)
    i0 = jnp.min(jnp.where(l >= m0, lane, 1000), axis=1, keepdims=True)
    sel0 = lane == i0
    l1 = jnp.where(sel0, -1e30, l)
    m1 = jnp.max(l1, axis=1, keepdims=True)
    i1 = jnp.min(jnp.where(l1 >= m1, lane, 1000), axis=1, keepdims=True)
    sel1 = lane == i1
    w0 = 1.0 / (1.0 + jnp.exp(m1 - m0))
    w1 = 1.0 - w0
    cmb_ref[...] = jnp.where(sel0, w0, 0.0) + jnp.where(sel1, w1, 0.0)


def _route(logits, sb=512):
    n = S // sb
    return pl.pallas_call(
        _route_body,
        grid=(n,),
        in_specs=[pl.BlockSpec((sb, 128), lambda i: (i, 0))],
        out_specs=pl.BlockSpec((sb, 128), lambda i: (i, 0)),
        out_shape=jax.ShapeDtypeStruct((S, 128), f32),
    )(logits)


def _moe_body(xn_ref, gw_ref, uw_ref, dw_ref, cmb_ref, hs_ref, out_ref):
    e = pl.program_id(1)
    x = xn_ref[...]
    g = jnp.dot(x, gw_ref[0], preferred_element_type=f32)
    u = jnp.dot(x, uw_ref[0], preferred_element_type=f32)
    a = ((g / (1.0 + jnp.exp(-g))) * u).astype(bf16)
    y = jnp.dot(a, dw_ref[0], preferred_element_type=f32)
    lane = jax.lax.broadcasted_iota(jnp.int32, cmb_ref.shape, 1)
    we = jnp.sum(jnp.where(lane == e, cmb_ref[...], 0.0), axis=1,
                 keepdims=True)
    contrib = we * y

    @pl.when(e == 0)
    def _():
        out_ref[...] = hs_ref[...] + contrib

    @pl.when(e != 0)
    def _():
        out_ref[...] += contrib


def _moe_dense(xn_bf, gate_w, up_w, down_w, cmb, hs2d, mb=512):
    n = S // mb
    return pl.pallas_call(
        _moe_body,
        grid=(n, E),
        in_specs=[
            pl.BlockSpec((mb, D), lambda i, e: (i, 0)),
            pl.BlockSpec((1, D, FF), lambda i, e: (e, 0, 0)),
            pl.BlockSpec((1, D, FF), lambda i, e: (e, 0, 0)),
            pl.BlockSpec((1, FF, D), lambda i, e: (e, 0, 0)),
            pl.BlockSpec((mb, 128), lambda i, e: (i, 0)),
            pl.BlockSpec((mb, D), lambda i, e: (i, 0)),
        ],
        out_specs=pl.BlockSpec((mb, D), lambda i, e: (i, 0)),
        out_shape=jax.ShapeDtypeStruct((S, D), f32),
    )(xn_bf, gate_w.astype(bf16), up_w.astype(bf16), down_w.astype(bf16),
      cmb, hs2d)


def kernel(hidden_states, position_ids, input_ln_w, q_w, k_w, v_w, o_w,
           q_ln_w, k_ln_w, post_ln_w, router_w, gate_w, up_w, down_w):
    h2d = hidden_states.reshape(S, D)
    pos = position_ids.reshape(S).astype(f32)
    inv = 1.0 / (THETA ** (jnp.arange(0, HD, 2, dtype=f32) / HD))
    ang = pos[:, None] * inv[None, :]
    cos_t = jnp.concatenate([jnp.cos(ang), jnp.cos(ang)], axis=1)
    sin_t = jnp.concatenate([jnp.sin(ang), jnp.sin(ang)], axis=1)

    q2d, k2d, v2d = _qkv(h2d, input_ln_w, q_w, k_w, v_w)
    kr2d = _krope(k2d, cos_t, sin_t, k_ln_w)
    o2d = _attention(q2d, kr2d, v2d, cos_t, sin_t, q_ln_w)

    rw_pad = jnp.pad(router_w, ((0, 0), (0, 128 - E))).astype(bf16)
    hs2d, xn_bf, logits = _oproj(o2d, o_w, h2d, post_ln_w, rw_pad)
    cmb = _route(logits)
    out = _moe_dense(xn_bf, gate_w, up_w, down_w, cmb, hs2d)
    return out.reshape(B, S, D)
